# Initial kernel scaffold; baseline (speedup 1.0000x reference)
import jax
import jax.numpy as jnp
from jax.experimental import pallas as pl


def kernel(x, edge_index, edge_weight, W1, b1, W2, b2):
    raise NotImplementedError("write your pallas kernel here")


# speedup vs baseline: 8.4823x; 8.4823x over previous
import functools

import jax
import jax.numpy as jnp
from jax import lax
from jax.experimental import pallas as pl
from jax.experimental.pallas import tpu as pltpu
from jax.experimental.pallas import tpu_sc as plsc

NC = 2
NS = 16
L = 16
NW = NC * NS

_SC_PARAMS = pltpu.CompilerParams(needs_layout_passes=False,
                                 use_tc_tiling_on_sc=False)


def _zero_chunk(NP):
    rpt = NP // NS
    for zc in (128, 64, 32, 16, 8):
        if rpt % zc == 0:
            return zc
    return 8


def _zero_acc(zb_v, acc_sh, s, *, NP, QD):
    RPT = NP // NS
    ZC = zb_v.shape[0]
    z = jnp.zeros((L,), jnp.float32)

    def _zb(i, carry):
        for q in range(QD):
            zb_v[i, pl.ds(q * L, L)] = z
        return carry
    lax.fori_loop(0, ZC, _zb, 0)

    def _za(k, carry):
        pltpu.sync_copy(zb_v, acc_sh.at[pl.ds(s * RPT + k * ZC, ZC)])
        return carry
    lax.fori_loop(0, RPT // ZC, _za, 0)


@functools.lru_cache(maxsize=None)
def _feat_agg(NP, E, D, B):
    DH = D // 2
    NB = E // (NS * B)
    assert NB * NS * B == E and NB % 8 == 0 and DH % L == 0 and B <= 128
    T = NB * B
    ZC = _zero_chunk(NP)
    QD = DH // L
    mesh = plsc.VectorSubcoreMesh(core_axis_name="c", subcore_axis_name="s")

    def body(g_hbm, src_h, dst_h, w_h, out_hbm,
             src_v, dst_v, w_v, rows_v, zb_v, acc_sh, sem):
        c = lax.axis_index("c")
        s = lax.axis_index("s")
        _zero_acc(zb_v, acc_sh, s, NP=NP, QD=QD)
        plsc.subcore_barrier()

        pltpu.sync_copy(src_h.at[pl.ds(s * NB, NB)], src_v)
        pltpu.sync_copy(dst_h.at[pl.ds(s * NB, NB)], dst_v)
        pltpu.sync_copy(w_h.at[pl.ds(s * T, T)], w_v)

        def _batch(j, carry):
            pltpu.async_copy(g_hbm.at[c].at[src_v.at[j]], rows_v, sem).wait()

            def _edge(e, c2):
                idx = jnp.full((L,), j * B + e, dtype=jnp.int32)
                wv = plsc.load_gather(w_v, [idx])
                for q in range(QD):
                    rows_v[e, pl.ds(q * L, L)] = rows_v[e, pl.ds(q * L, L)] * wv
                return c2
            lax.fori_loop(0, B, _edge, 0)

            pltpu.sync_copy(rows_v, acc_sh.at[dst_v.at[j]], add=True)
            return carry
        lax.fori_loop(0, NB, _batch, 0)

        plsc.subcore_barrier()
        RPT = NP // NS
        pltpu.sync_copy(acc_sh.at[pl.ds(s * RPT, RPT)],
                        out_hbm.at[c, pl.ds(s * RPT, RPT)])

    return pl.kernel(
        body,
        out_type=jax.ShapeDtypeStruct((NC, NP, DH), jnp.float32),
        mesh=mesh,
        compiler_params=_SC_PARAMS,
        scratch_types=[
            pltpu.VMEM((NB, B), jnp.int32),
            pltpu.VMEM((NB, B), jnp.int32),
            pltpu.VMEM((T,), jnp.float32),
            pltpu.VMEM((B, DH), jnp.float32),
            pltpu.VMEM((ZC, DH), jnp.float32),
            pltpu.VMEM_SHARED((NP, DH), jnp.float32),
            pltpu.SemaphoreType.DMA,
        ],
    )


@functools.lru_cache(maxsize=None)
def _deg_agg(NP, E, B):
    D = L
    NB = E // (NW * B)
    assert NB * NW * B == E and NB % 8 == 0
    T = NB * B
    ZC = _zero_chunk(NP)
    mesh = plsc.VectorSubcoreMesh(core_axis_name="c", subcore_axis_name="s")

    def body(dst_h, w_h, out_hbm, dst_v, w_v, rows_v, zb_v, acc_sh, sem):
        c = lax.axis_index("c")
        s = lax.axis_index("s")
        wid = c * NS + s
        _zero_acc(zb_v, acc_sh, s, NP=NP, QD=1)
        plsc.subcore_barrier()

        pltpu.sync_copy(dst_h.at[pl.ds(wid * NB, NB)], dst_v)
        pltpu.sync_copy(w_h.at[pl.ds(wid * T, T)], w_v)

        def _batch(j, carry):
            def _edge(e, c2):
                idx = jnp.full((L,), j * B + e, dtype=jnp.int32)
                rows_v[e, pl.ds(0, L)] = plsc.load_gather(w_v, [idx])
                return c2
            lax.fori_loop(0, B, _edge, 0)
            pltpu.sync_copy(rows_v, acc_sh.at[dst_v.at[j]], add=True)
            return carry
        lax.fori_loop(0, NB, _batch, 0)

        plsc.subcore_barrier()
        RPT = NP // NS
        pltpu.sync_copy(acc_sh.at[pl.ds(s * RPT, RPT)],
                        out_hbm.at[c, pl.ds(s * RPT, RPT)])

    return pl.kernel(
        body,
        out_type=jax.ShapeDtypeStruct((NC, NP, D), jnp.float32),
        mesh=mesh,
        compiler_params=_SC_PARAMS,
        scratch_types=[
            pltpu.VMEM((NB, B), jnp.int32),
            pltpu.VMEM((T,), jnp.float32),
            pltpu.VMEM((B, D), jnp.float32),
            pltpu.VMEM((ZC, D), jnp.float32),
            pltpu.VMEM_SHARED((NP, D), jnp.float32),
            pltpu.SemaphoreType.DMA,
        ],
    )


def _deg_stats(degp_ref):
    deg = 1.0 + degp_ref[0, :, 0] + degp_ref[1, :, 0]
    return lax.rsqrt(deg), 1.0 / deg


def _tc1_body(x_ref, w1_ref, degp_ref, g1_ref, s1_ref):
    h = jnp.dot(x_ref[...], w1_ref[...], preferred_element_type=jnp.float32)
    dis, invd = _deg_stats(degp_ref)
    g = h * dis[:, None]
    DH = g.shape[1] // 2
    g1_ref[0] = g[:, :DH]
    g1_ref[1] = g[:, DH:]
    s1_ref[...] = h * invd[:, None]


def _tc2_body(p1_ref, s1_ref, degp_ref, b1_ref, w2_ref, g2_ref, s2_ref):
    dis, invd = _deg_stats(degp_ref)
    p1 = jnp.concatenate([p1_ref[0], p1_ref[1]], axis=1)
    h1 = jnp.maximum(p1 * dis[:, None] + s1_ref[...] + b1_ref[...], 0.0)
    h2 = jnp.dot(h1, w2_ref[...], preferred_element_type=jnp.float32)
    g = h2 * dis[:, None]
    DH = g.shape[1] // 2
    g2_ref[0] = g[:, :DH]
    g2_ref[1] = g[:, DH:]
    s2_ref[...] = h2 * invd[:, None]


def _tc3_body(p2_ref, s2_ref, degp_ref, b2_ref, out_ref, *, C):
    dis, _ = _deg_stats(degp_ref)
    p2 = jnp.concatenate([p2_ref[0], p2_ref[1]], axis=1)
    o = p2 * dis[:, None] + s2_ref[...] + b2_ref[...]
    col = lax.broadcasted_iota(jnp.int32, o.shape, 1)
    om = jnp.where(col < C, o, jnp.float32(-1e30))
    m = jnp.max(om, axis=1, keepdims=True)
    lse = jnp.log(jnp.sum(jnp.exp(om - m), axis=1, keepdims=True)) + m
    out_ref[...] = (om - lse)[:, :C]


def kernel(x, edge_index, edge_weight, W1, b1, W2, b2):
    N, F = x.shape
    H = W1.shape[1]
    C = W2.shape[1]
    E = edge_weight.shape[0]
    CP = ((C + 2 * L - 1) // (2 * L)) * (2 * L)
    HH, CH = H // 2, CP // 2

    NP = ((N + 8 * NS - 1) // (8 * NS)) * (8 * NS)
    B = 50
    assert E % (NW * B) == 0

    src2 = edge_index[0].reshape(E // B, B)
    dst2 = edge_index[1].reshape(E // B, B)
    wflat = edge_weight

    W2p = jnp.pad(W2, ((0, 0), (0, CP - C)))
    b1r = b1.reshape(1, H)
    b2p = jnp.pad(b2, (0, CP - C)).reshape(1, CP)

    degp = _deg_agg(NP, E, B)(dst2, wflat)

    R = 1000
    grid = (N // R,)
    f32 = jnp.float32

    g1, s1 = pl.pallas_call(
        _tc1_body,
        grid=grid,
        in_specs=[pl.BlockSpec((R, F), lambda i: (i, 0)),
                  pl.BlockSpec((F, H), lambda i: (0, 0)),
                  pl.BlockSpec((NC, R, L), lambda i: (0, i, 0))],
        out_specs=[pl.BlockSpec((NC, R, HH), lambda i: (0, i, 0)),
                   pl.BlockSpec((R, H), lambda i: (i, 0))],
        out_shape=[jax.ShapeDtypeStruct((NC, N, HH), f32),
                   jax.ShapeDtypeStruct((N, H), f32)],
    )(x, W1, degp)

    p1 = _feat_agg(NP, E, H, B)(g1, src2, dst2, wflat)

    g2, s2 = pl.pallas_call(
        _tc2_body,
        grid=grid,
        in_specs=[pl.BlockSpec((NC, R, HH), lambda i: (0, i, 0)),
                  pl.BlockSpec((R, H), lambda i: (i, 0)),
                  pl.BlockSpec((NC, R, L), lambda i: (0, i, 0)),
                  pl.BlockSpec((1, H), lambda i: (0, 0)),
                  pl.BlockSpec((H, CP), lambda i: (0, 0))],
        out_specs=[pl.BlockSpec((NC, R, CH), lambda i: (0, i, 0)),
                   pl.BlockSpec((R, CP), lambda i: (i, 0))],
        out_shape=[jax.ShapeDtypeStruct((NC, N, CH), f32),
                   jax.ShapeDtypeStruct((N, CP), f32)],
    )(p1, s1, degp, b1r, W2p)

    p2 = _feat_agg(NP, E, CP, B)(g2, src2, dst2, wflat)

    out = pl.pallas_call(
        functools.partial(_tc3_body, C=C),
        grid=grid,
        in_specs=[pl.BlockSpec((NC, R, CH), lambda i: (0, i, 0)),
                  pl.BlockSpec((R, CP), lambda i: (i, 0)),
                  pl.BlockSpec((NC, R, L), lambda i: (0, i, 0)),
                  pl.BlockSpec((1, CP), lambda i: (0, 0))],
        out_specs=pl.BlockSpec((R, C), lambda i: (i, 0)),
        out_shape=jax.ShapeDtypeStruct((N, C), f32),
    )(p2, s2, degp, b2p)

    return out


# speedup vs baseline: 19.3069x; 2.2761x over previous
import functools

import jax
import jax.numpy as jnp
from jax import lax
from jax.experimental import pallas as pl
from jax.experimental.pallas import tpu as pltpu
from jax.experimental.pallas import tpu_sc as plsc

NC = 2
NS = 16
L = 16
NW = NC * NS

_SC_PARAMS = pltpu.CompilerParams(needs_layout_passes=False,
                                 use_tc_tiling_on_sc=False)


def _zero_chunk(NP):
    rpt = NP // NS
    for zc in (128, 64, 32, 16, 8):
        if rpt % zc == 0:
            return zc
    return 8


def _zero_acc(zb_v, acc_sh, s, *, NP, QD):
    RPT = NP // NS
    ZC = zb_v.shape[0]
    z = jnp.zeros((L,), jnp.float32)

    def _zb(i, carry):
        for q in range(QD):
            zb_v[i, pl.ds(q * L, L)] = z
        return carry
    lax.fori_loop(0, ZC, _zb, 0)

    def _za(k, carry):
        pltpu.sync_copy(zb_v, acc_sh.at[pl.ds(s * RPT + k * ZC, ZC)])
        return carry
    lax.fori_loop(0, RPT // ZC, _za, 0)


@functools.lru_cache(maxsize=None)
def _feat_agg(NP, E, D, B):
    DH = D // 2
    NB = E // (NS * B)
    assert NB * NS * B == E and NB % 8 == 0 and DH % L == 0 and B <= 128
    T = NB * B
    ZC = _zero_chunk(NP)
    QD = DH // L
    mesh = plsc.VectorSubcoreMesh(core_axis_name="c", subcore_axis_name="s")

    NBUF = 4
    assert NB % NBUF == 0 and NB >= 2 * NBUF
    UN = 5 if B % 5 == 0 else 1

    def body(g_hbm, src_h, dst_h, w_h, out_hbm,
             src_v, dst_v, w_v, rows_v, zb_v, acc_sh, *sems):
        gs, ss = sems[:NBUF], sems[NBUF:]
        c = lax.axis_index("c")
        s = lax.axis_index("s")
        _zero_acc(zb_v, acc_sh, s, NP=NP, QD=QD)
        plsc.subcore_barrier()

        pltpu.sync_copy(src_h.at[pl.ds(s * NB, NB)], src_v)
        pltpu.sync_copy(dst_h.at[pl.ds(s * NB, NB)], dst_v)
        pltpu.sync_copy(w_h.at[pl.ds(s * T, T)], w_v)

        def gat(jj, b):
            return pltpu.async_copy(g_hbm.at[c].at[src_v.at[jj]],
                                    rows_v.at[b], gs[b])

        def gat_wait(jj, b):
            pltpu.make_async_copy(g_hbm.at[c].at[src_v.at[jj]],
                                  rows_v.at[b], gs[b]).wait()

        def sca(jj, b):
            return pltpu.async_copy(rows_v.at[b], acc_sh.at[dst_v.at[jj]],
                                    ss[b], add=True)

        def sca_wait(jj, b):
            pltpu.make_async_copy(rows_v.at[b], acc_sh.at[dst_v.at[jj]],
                                  ss[b]).wait()

        gat(0, 0)
        gat(1, 1)

        def _group(g, carry):
            for u in range(NBUF):
                jj = g * NBUF + u
                gat_wait(jj, u)

                def _scl(i, c2):
                    for v in range(UN):
                        e = i * UN + v
                        idx = jnp.full((L,), jj * B + e, dtype=jnp.int32)
                        wv = plsc.load_gather(w_v, [idx])
                        for q in range(QD):
                            rows_v[u, e, pl.ds(q * L, L)] = (
                                rows_v[u, e, pl.ds(q * L, L)] * wv)
                    return c2
                lax.fori_loop(0, B // UN, _scl, 0)

                @pl.when(jj >= 1)
                def _():
                    sca_wait(jj - 1, (u + NBUF - 1) % NBUF)

                @pl.when(jj + 2 < NB)
                def _():
                    gat(jj + 2, (u + 2) % NBUF)

                sca(jj, u)
            return carry
        lax.fori_loop(0, NB // NBUF, _group, 0)
        sca_wait(NB - 1, NBUF - 1)

        plsc.subcore_barrier()
        RPT = NP // NS
        pltpu.sync_copy(acc_sh.at[pl.ds(s * RPT, RPT)],
                        out_hbm.at[c, pl.ds(s * RPT, RPT)])

    return pl.kernel(
        body,
        out_type=jax.ShapeDtypeStruct((NC, NP, DH), jnp.float32),
        mesh=mesh,
        compiler_params=_SC_PARAMS,
        scratch_types=[
            pltpu.VMEM((NB, B), jnp.int32),
            pltpu.VMEM((NB, B), jnp.int32),
            pltpu.VMEM((T,), jnp.float32),
            pltpu.VMEM((NBUF, B, DH), jnp.float32),
            pltpu.VMEM((ZC, DH), jnp.float32),
            pltpu.VMEM_SHARED((NP, DH), jnp.float32),
        ] + [pltpu.SemaphoreType.DMA] * (2 * NBUF),
    )


@functools.lru_cache(maxsize=None)
def _deg_agg(NP, E, B):
    D = L
    NB = E // (NW * B)
    assert NB * NW * B == E and NB % 8 == 0
    T = NB * B
    ZC = _zero_chunk(NP)
    mesh = plsc.VectorSubcoreMesh(core_axis_name="c", subcore_axis_name="s")

    def body(dst_h, w_h, out_hbm, dst_v, w_v, rows_v, zb_v, acc_sh, sem):
        c = lax.axis_index("c")
        s = lax.axis_index("s")
        wid = c * NS + s
        _zero_acc(zb_v, acc_sh, s, NP=NP, QD=1)
        plsc.subcore_barrier()

        pltpu.sync_copy(dst_h.at[pl.ds(wid * NB, NB)], dst_v)
        pltpu.sync_copy(w_h.at[pl.ds(wid * T, T)], w_v)

        def _batch(j, carry):
            def _edge(e, c2):
                idx = jnp.full((L,), j * B + e, dtype=jnp.int32)
                rows_v[e, pl.ds(0, L)] = plsc.load_gather(w_v, [idx])
                return c2
            lax.fori_loop(0, B, _edge, 0)
            pltpu.sync_copy(rows_v, acc_sh.at[dst_v.at[j]], add=True)
            return carry
        lax.fori_loop(0, NB, _batch, 0)

        plsc.subcore_barrier()
        RPT = NP // NS
        pltpu.sync_copy(acc_sh.at[pl.ds(s * RPT, RPT)],
                        out_hbm.at[c, pl.ds(s * RPT, RPT)])

    return pl.kernel(
        body,
        out_type=jax.ShapeDtypeStruct((NC, NP, D), jnp.float32),
        mesh=mesh,
        compiler_params=_SC_PARAMS,
        scratch_types=[
            pltpu.VMEM((NB, B), jnp.int32),
            pltpu.VMEM((T,), jnp.float32),
            pltpu.VMEM((B, D), jnp.float32),
            pltpu.VMEM((ZC, D), jnp.float32),
            pltpu.VMEM_SHARED((NP, D), jnp.float32),
            pltpu.SemaphoreType.DMA,
        ],
    )


def _deg_stats(degp_ref):
    deg = 1.0 + degp_ref[0, :, 0] + degp_ref[1, :, 0]
    return lax.rsqrt(deg), 1.0 / deg


def _tc1_body(x_ref, w1_ref, degp_ref, g1_ref, s1_ref):
    h = jnp.dot(x_ref[...], w1_ref[...], preferred_element_type=jnp.float32)
    dis, invd = _deg_stats(degp_ref)
    g = h * dis[:, None]
    DH = g.shape[1] // 2
    g1_ref[0] = g[:, :DH]
    g1_ref[1] = g[:, DH:]
    s1_ref[...] = h * invd[:, None]


def _tc2_body(p1_ref, s1_ref, degp_ref, b1_ref, w2_ref, g2_ref, s2_ref):
    dis, invd = _deg_stats(degp_ref)
    p1 = jnp.concatenate([p1_ref[0], p1_ref[1]], axis=1)
    h1 = jnp.maximum(p1 * dis[:, None] + s1_ref[...] + b1_ref[...], 0.0)
    h2 = jnp.dot(h1, w2_ref[...], preferred_element_type=jnp.float32)
    g = h2 * dis[:, None]
    DH = g.shape[1] // 2
    g2_ref[0] = g[:, :DH]
    g2_ref[1] = g[:, DH:]
    s2_ref[...] = h2 * invd[:, None]


def _tc3_body(p2_ref, s2_ref, degp_ref, b2_ref, out_ref, *, C):
    dis, _ = _deg_stats(degp_ref)
    p2 = jnp.concatenate([p2_ref[0], p2_ref[1]], axis=1)
    o = p2 * dis[:, None] + s2_ref[...] + b2_ref[...]
    col = lax.broadcasted_iota(jnp.int32, o.shape, 1)
    om = jnp.where(col < C, o, jnp.float32(-1e30))
    m = jnp.max(om, axis=1, keepdims=True)
    lse = jnp.log(jnp.sum(jnp.exp(om - m), axis=1, keepdims=True)) + m
    out_ref[...] = (om - lse)[:, :C]


def kernel(x, edge_index, edge_weight, W1, b1, W2, b2):
    N, F = x.shape
    H = W1.shape[1]
    C = W2.shape[1]
    E = edge_weight.shape[0]
    CP = ((C + 2 * L - 1) // (2 * L)) * (2 * L)
    HH, CH = H // 2, CP // 2

    NP = ((N + 8 * NS - 1) // (8 * NS)) * (8 * NS)
    B = 100
    assert E % (NW * B) == 0

    src2 = edge_index[0].reshape(E // B, B)
    dst2 = edge_index[1].reshape(E // B, B)
    wflat = edge_weight

    W2p = jnp.pad(W2, ((0, 0), (0, CP - C)))
    b1r = b1.reshape(1, H)
    b2p = jnp.pad(b2, (0, CP - C)).reshape(1, CP)

    BD = 125
    dst2d = edge_index[1].reshape(E // BD, BD)
    degp = _deg_agg(NP, E, BD)(dst2d, wflat)

    R = 1000
    grid = (N // R,)
    f32 = jnp.float32

    g1, s1 = pl.pallas_call(
        _tc1_body,
        grid=grid,
        in_specs=[pl.BlockSpec((R, F), lambda i: (i, 0)),
                  pl.BlockSpec((F, H), lambda i: (0, 0)),
                  pl.BlockSpec((NC, R, L), lambda i: (0, i, 0))],
        out_specs=[pl.BlockSpec((NC, R, HH), lambda i: (0, i, 0)),
                   pl.BlockSpec((R, H), lambda i: (i, 0))],
        out_shape=[jax.ShapeDtypeStruct((NC, N, HH), f32),
                   jax.ShapeDtypeStruct((N, H), f32)],
    )(x, W1, degp)

    p1 = _feat_agg(NP, E, H, B)(g1, src2, dst2, wflat)

    g2, s2 = pl.pallas_call(
        _tc2_body,
        grid=grid,
        in_specs=[pl.BlockSpec((NC, R, HH), lambda i: (0, i, 0)),
                  pl.BlockSpec((R, H), lambda i: (i, 0)),
                  pl.BlockSpec((NC, R, L), lambda i: (0, i, 0)),
                  pl.BlockSpec((1, H), lambda i: (0, 0)),
                  pl.BlockSpec((H, CP), lambda i: (0, 0))],
        out_specs=[pl.BlockSpec((NC, R, CH), lambda i: (0, i, 0)),
                   pl.BlockSpec((R, CP), lambda i: (i, 0))],
        out_shape=[jax.ShapeDtypeStruct((NC, N, CH), f32),
                   jax.ShapeDtypeStruct((N, CP), f32)],
    )(p1, s1, degp, b1r, W2p)

    p2 = _feat_agg(NP, E, CP, B)(g2, src2, dst2, wflat)

    out = pl.pallas_call(
        functools.partial(_tc3_body, C=C),
        grid=grid,
        in_specs=[pl.BlockSpec((NC, R, CH), lambda i: (0, i, 0)),
                  pl.BlockSpec((R, CP), lambda i: (i, 0)),
                  pl.BlockSpec((NC, R, L), lambda i: (0, i, 0)),
                  pl.BlockSpec((1, CP), lambda i: (0, 0))],
        out_specs=pl.BlockSpec((R, C), lambda i: (i, 0)),
        out_shape=jax.ShapeDtypeStruct((N, C), f32),
    )(p2, s2, degp, b2p)

    return out


# speedup vs baseline: 24.9261x; 1.2910x over previous
import functools

import jax
import jax.numpy as jnp
from jax import lax
from jax.experimental import pallas as pl
from jax.experimental.pallas import tpu as pltpu
from jax.experimental.pallas import tpu_sc as plsc

NC = 2
NS = 16
L = 16
NW = NC * NS

_SC_PARAMS = pltpu.CompilerParams(needs_layout_passes=False,
                                 use_tc_tiling_on_sc=False)


def _zero_chunk(NP):
    rpt = NP // NS
    for zc in (128, 64, 32, 16, 8):
        if rpt % zc == 0:
            return zc
    return 8


def _zero_acc(zb_v, acc_sh, s, *, NP, QD):
    RPT = NP // NS
    ZC = zb_v.shape[0]
    z = jnp.zeros((L,), jnp.float32)

    def _zb(i, carry):
        for q in range(QD):
            zb_v[i, pl.ds(q * L, L)] = z
        return carry
    lax.fori_loop(0, ZC, _zb, 0)

    def _za(k, carry):
        pltpu.sync_copy(zb_v, acc_sh.at[pl.ds(s * RPT + k * ZC, ZC)])
        return carry
    lax.fori_loop(0, RPT // ZC, _za, 0)


@functools.lru_cache(maxsize=None)
def _feat_agg(NP, E, D, B):
    DH = D // 2
    NB = E // (NS * B)
    assert NB * NS * B == E and DH % L == 0 and B <= 128
    T = NB * B
    ZC = _zero_chunk(NP)
    QD = DH // L
    mesh = plsc.VectorSubcoreMesh(core_axis_name="c", subcore_axis_name="s")
    NBUF = 4
    assert NB % NBUF == 0 and NB >= 2 * NBUF

    def body(g_hbm, src_h, dst_h, w_h, out_hbm,
             src_v, dst_v, w_v, rows_v, zb_v, acc_sh, *sems):
        gs, ss = sems[:NBUF], sems[NBUF:]
        c = lax.axis_index("c")
        s = lax.axis_index("s")
        _zero_acc(zb_v, acc_sh, s, NP=NP, QD=QD)
        plsc.subcore_barrier()

        pltpu.sync_copy(src_h.at[pl.ds(s * NB, NB)], src_v)
        pltpu.sync_copy(dst_h.at[pl.ds(s * NB, NB)], dst_v)
        pltpu.sync_copy(w_h.at[pl.ds(s * T, T)], w_v)

        def gat(jj, b):
            return pltpu.async_copy(g_hbm.at[c].at[src_v.at[jj]],
                                    rows_v.at[b], gs[b])

        def gat_wait(jj, b):
            pltpu.make_async_copy(g_hbm.at[c].at[src_v.at[jj]],
                                  rows_v.at[b], gs[b]).wait()

        def sca(jj, b):
            return pltpu.async_copy(rows_v.at[b], acc_sh.at[dst_v.at[jj]],
                                    ss[b], add=True)

        def sca_wait(jj, b):
            pltpu.make_async_copy(rows_v.at[b], acc_sh.at[dst_v.at[jj]],
                                  ss[b]).wait()

        gat(0, 0)
        gat(1, 1)

        def _group(g, carry):
            for u in range(NBUF):
                jj = g * NBUF + u
                gat_wait(jj, u)

                @plsc.parallel_loop(0, B, step=1, unroll=4)
                def _scl(e):
                    idx = jnp.full((L,), jj * B + e, dtype=jnp.int32)
                    wv = plsc.load_gather(w_v, [idx])
                    for q in range(QD):
                        rows_v[u, e, pl.ds(q * L, L)] = (
                            rows_v[u, e, pl.ds(q * L, L)] * wv)

                @pl.when(jj >= 1)
                def _():
                    sca_wait(jj - 1, (u + NBUF - 1) % NBUF)

                @pl.when(jj + 2 < NB)
                def _():
                    gat(jj + 2, (u + 2) % NBUF)

                sca(jj, u)
            return carry
        lax.fori_loop(0, NB // NBUF, _group, 0)
        sca_wait(NB - 1, NBUF - 1)

        plsc.subcore_barrier()
        RPT = NP // NS
        pltpu.sync_copy(acc_sh.at[pl.ds(s * RPT, RPT)],
                        out_hbm.at[c, pl.ds(s * RPT, RPT)])

    return pl.kernel(
        body,
        out_type=jax.ShapeDtypeStruct((NC, NP, DH), jnp.float32),
        mesh=mesh,
        compiler_params=_SC_PARAMS,
        scratch_types=[
            pltpu.VMEM((NB, B), jnp.int32),
            pltpu.VMEM((NB, B), jnp.int32),
            pltpu.VMEM((T,), jnp.float32),
            pltpu.VMEM((NBUF, B, DH), jnp.float32),
            pltpu.VMEM((ZC, DH), jnp.float32),
            pltpu.VMEM_SHARED((NP, DH), jnp.float32),
        ] + [pltpu.SemaphoreType.DMA] * (2 * NBUF),
    )


@functools.lru_cache(maxsize=None)
def _deg_agg(NP, E, B):
    D = L
    NB = E // (NW * B)
    assert NB * NW * B == E
    T = NB * B
    ZC = _zero_chunk(NP)
    mesh = plsc.VectorSubcoreMesh(core_axis_name="c", subcore_axis_name="s")
    NBUF = 2
    assert NB % NBUF == 0

    def body(dst_h, w_h, out_hbm, dst_v, w_v, rows_v, zb_v, acc_sh, *sems):
        c = lax.axis_index("c")
        s = lax.axis_index("s")
        wid = c * NS + s
        _zero_acc(zb_v, acc_sh, s, NP=NP, QD=1)
        plsc.subcore_barrier()

        pltpu.sync_copy(dst_h.at[pl.ds(wid * NB, NB)], dst_v)
        pltpu.sync_copy(w_h.at[pl.ds(wid * T, T)], w_v)

        def _group(g, carry):
            for u in range(NBUF):
                jj = g * NBUF + u

                @pl.when(jj >= NBUF)
                def _():
                    pltpu.make_async_copy(
                        rows_v.at[u], acc_sh.at[dst_v.at[jj - NBUF]],
                        sems[u]).wait()

                @plsc.parallel_loop(0, B, step=1, unroll=4)
                def _bld(e):
                    idx = jnp.full((L,), jj * B + e, dtype=jnp.int32)
                    rows_v[u, e, pl.ds(0, L)] = plsc.load_gather(w_v, [idx])

                pltpu.async_copy(rows_v.at[u], acc_sh.at[dst_v.at[jj]],
                                 sems[u], add=True)
            return carry
        lax.fori_loop(0, NB // NBUF, _group, 0)
        for u in range(NBUF):
            pltpu.make_async_copy(rows_v.at[u],
                                  acc_sh.at[dst_v.at[NB - NBUF + u]],
                                  sems[u]).wait()

        plsc.subcore_barrier()
        RPT = NP // NS
        pltpu.sync_copy(acc_sh.at[pl.ds(s * RPT, RPT)],
                        out_hbm.at[c, pl.ds(s * RPT, RPT)])

    return pl.kernel(
        body,
        out_type=jax.ShapeDtypeStruct((NC, NP, D), jnp.float32),
        mesh=mesh,
        compiler_params=_SC_PARAMS,
        scratch_types=[
            pltpu.VMEM((NB, B), jnp.int32),
            pltpu.VMEM((T,), jnp.float32),
            pltpu.VMEM((NBUF, B, D), jnp.float32),
            pltpu.VMEM((ZC, D), jnp.float32),
            pltpu.VMEM_SHARED((NP, D), jnp.float32),
        ] + [pltpu.SemaphoreType.DMA] * NBUF,
    )


def _deg_stats(degp_ref):
    deg = 1.0 + degp_ref[0, :, 0] + degp_ref[1, :, 0]
    return lax.rsqrt(deg), 1.0 / deg


def _tc1_body(x_ref, w1_ref, degp_ref, g1_ref, s1_ref):
    h = jnp.dot(x_ref[...], w1_ref[...], preferred_element_type=jnp.float32)
    dis, invd = _deg_stats(degp_ref)
    g = h * dis[:, None]
    DH = g.shape[1] // 2
    g1_ref[0] = g[:, :DH]
    g1_ref[1] = g[:, DH:]
    s1_ref[...] = h * invd[:, None]


def _tc2_body(p1_ref, s1_ref, degp_ref, b1_ref, w2_ref, g2_ref, s2_ref):
    dis, invd = _deg_stats(degp_ref)
    p1 = jnp.concatenate([p1_ref[0], p1_ref[1]], axis=1)
    h1 = jnp.maximum(p1 * dis[:, None] + s1_ref[...] + b1_ref[...], 0.0)
    h2 = jnp.dot(h1, w2_ref[...], preferred_element_type=jnp.float32)
    g = h2 * dis[:, None]
    DH = g.shape[1] // 2
    g2_ref[0] = g[:, :DH]
    g2_ref[1] = g[:, DH:]
    s2_ref[...] = h2 * invd[:, None]


def _tc3_body(p2_ref, s2_ref, degp_ref, b2_ref, out_ref, *, C):
    dis, _ = _deg_stats(degp_ref)
    p2 = jnp.concatenate([p2_ref[0], p2_ref[1]], axis=1)
    o = p2 * dis[:, None] + s2_ref[...] + b2_ref[...]
    col = lax.broadcasted_iota(jnp.int32, o.shape, 1)
    om = jnp.where(col < C, o, jnp.float32(-1e30))
    m = jnp.max(om, axis=1, keepdims=True)
    lse = jnp.log(jnp.sum(jnp.exp(om - m), axis=1, keepdims=True)) + m
    out_ref[...] = (om - lse)[:, :C]


def kernel(x, edge_index, edge_weight, W1, b1, W2, b2):
    N, F = x.shape
    H = W1.shape[1]
    C = W2.shape[1]
    E = edge_weight.shape[0]
    CP = ((C + 2 * L - 1) // (2 * L)) * (2 * L)
    HH, CH = H // 2, CP // 2

    NP = ((N + 8 * NS - 1) // (8 * NS)) * (8 * NS)
    B = 100
    assert E % (NW * B) == 0

    src2 = edge_index[0].reshape(E // B, B)
    dst2 = edge_index[1].reshape(E // B, B)
    wflat = edge_weight

    W2p = jnp.pad(W2, ((0, 0), (0, CP - C)))
    b1r = b1.reshape(1, H)
    b2p = jnp.pad(b2, (0, CP - C)).reshape(1, CP)

    BD = 100
    dst2d = edge_index[1].reshape(E // BD, BD)
    degp = _deg_agg(NP, E, BD)(dst2d, wflat)

    R = 1000
    grid = (N // R,)
    f32 = jnp.float32

    g1, s1 = pl.pallas_call(
        _tc1_body,
        grid=grid,
        in_specs=[pl.BlockSpec((R, F), lambda i: (i, 0)),
                  pl.BlockSpec((F, H), lambda i: (0, 0)),
                  pl.BlockSpec((NC, R, L), lambda i: (0, i, 0))],
        out_specs=[pl.BlockSpec((NC, R, HH), lambda i: (0, i, 0)),
                   pl.BlockSpec((R, H), lambda i: (i, 0))],
        out_shape=[jax.ShapeDtypeStruct((NC, N, HH), f32),
                   jax.ShapeDtypeStruct((N, H), f32)],
    )(x, W1, degp)

    p1 = _feat_agg(NP, E, H, B)(g1, src2, dst2, wflat)

    g2, s2 = pl.pallas_call(
        _tc2_body,
        grid=grid,
        in_specs=[pl.BlockSpec((NC, R, HH), lambda i: (0, i, 0)),
                  pl.BlockSpec((R, H), lambda i: (i, 0)),
                  pl.BlockSpec((NC, R, L), lambda i: (0, i, 0)),
                  pl.BlockSpec((1, H), lambda i: (0, 0)),
                  pl.BlockSpec((H, CP), lambda i: (0, 0))],
        out_specs=[pl.BlockSpec((NC, R, CH), lambda i: (0, i, 0)),
                   pl.BlockSpec((R, CP), lambda i: (i, 0))],
        out_shape=[jax.ShapeDtypeStruct((NC, N, CH), f32),
                   jax.ShapeDtypeStruct((N, CP), f32)],
    )(p1, s1, degp, b1r, W2p)

    p2 = _feat_agg(NP, E, CP, B)(g2, src2, dst2, wflat)

    out = pl.pallas_call(
        functools.partial(_tc3_body, C=C),
        grid=grid,
        in_specs=[pl.BlockSpec((NC, R, CH), lambda i: (0, i, 0)),
                  pl.BlockSpec((R, CP), lambda i: (i, 0)),
                  pl.BlockSpec((NC, R, L), lambda i: (0, i, 0)),
                  pl.BlockSpec((1, CP), lambda i: (0, 0))],
        out_specs=pl.BlockSpec((R, C), lambda i: (i, 0)),
        out_shape=jax.ShapeDtypeStruct((N, C), f32),
    )(p2, s2, degp, b2p)

    return out
